# baseline (speedup 1.0000x reference)
import functools

import jax
import jax.numpy as jnp
from jax import lax
from jax.experimental import pallas as pl
from jax.experimental.pallas import tpu as pltpu
from jax.experimental.pallas import tpu_sc as plsc

N = 10000
D = 64
K = 10
ALPHA = 0.1
NC, NS = 2, 16
NW = NC * NS
CH = 128
NCHUNK = 80
EP_TILE = CH * NCHUNK
E_PAD = EP_TILE * NW
N_PAD = 10016
ROWS_T = N_PAD // NS
PROWS_T = N // NS

NGRP = 2
NSUPER = NCHUNK // (2 * NGRP)
IDXH = NCHUNK // 2

_mesh = plsc.VectorSubcoreMesh(core_axis_name="c", subcore_axis_name="s")


MIXB = 125


def _make_sc_step(mix):
    def body(*refs):
        if mix:
            (scat_hbm, pprev_hbm, d2_hbm, hd_hbm, src_hbm, dst_hbm, zeros_hbm,
             out_hbm, pnew_hbm,
             sidx, didx, rA0, rA1, rB0, rB1, d2b,
             p_sh, raw_sh, gsemA, gsemB, ssemA, ssemB) = refs
        else:
            (p_hbm, src_hbm, dst_hbm, zeros_hbm, out_hbm,
             sidx, didx, rA0, rA1, rB0, rB1,
             p_sh, raw_sh, gsemA, gsemB, ssemA, ssemB) = refs
        c = lax.axis_index("c")
        s = lax.axis_index("s")
        wid = c * NS + s
        rA = [rA0, rA1]
        rB = [rB0, rB1]
        pltpu.sync_copy(src_hbm.at[wid, pl.ds(0, IDXH)], sidx)
        pltpu.sync_copy(dst_hbm.at[wid, pl.ds(0, IDXH)], didx)
        base = s * ROWS_T
        pltpu.sync_copy(zeros_hbm, raw_sh.at[pl.ds(base, ROWS_T)])
        if not mix:
            pltpu.sync_copy(p_hbm.at[pl.ds(s * PROWS_T, PROWS_T)],
                            p_sh.at[pl.ds(s * PROWS_T, PROWS_T)])
        else:
            for blk in range(PROWS_T // MIXB):
                r0 = s * PROWS_T + blk * MIXB
                hs = [
                    pltpu.async_copy(scat_hbm.at[0, pl.ds(r0, MIXB)],
                                     rA0.at[pl.ds(0, MIXB)], gsemA),
                    pltpu.async_copy(scat_hbm.at[1, pl.ds(r0, MIXB)],
                                     rA1.at[pl.ds(0, MIXB)], gsemA),
                    pltpu.async_copy(pprev_hbm.at[pl.ds(r0, MIXB)],
                                     rB0.at[pl.ds(0, MIXB)], gsemA),
                    pltpu.async_copy(hd_hbm.at[pl.ds(r0, MIXB)],
                                     rB1.at[pl.ds(0, MIXB)], gsemA),
                    pltpu.async_copy(d2_hbm.at[pl.ds(r0, MIXB)],
                                     d2b.at[pl.ds(0, MIXB)], gsemA),
                ]
                for hh in hs:
                    hh.wait()

                @plsc.parallel_loop(0, MIXB, unroll=5)
                def _(r):
                    dv = d2b[r, pl.ds(0, 16)]
                    for j in range(4):
                        jj = j * 16
                        rA0[r, pl.ds(jj, 16)] = (
                            (rA0[r, pl.ds(jj, 16)] + rA1[r, pl.ds(jj, 16)]
                             + rB0[r, pl.ds(jj, 16)]) * dv
                            + rB1[r, pl.ds(jj, 16)])
                pltpu.sync_copy(rA0.at[pl.ds(0, MIXB)],
                                p_sh.at[pl.ds(r0, MIXB)])

                @pl.when(c == (blk % 2))
                def _():
                    pltpu.sync_copy(rA0.at[pl.ds(0, MIXB)],
                                    pnew_hbm.at[pl.ds(r0, MIXB)])
        plsc.subcore_barrier()

        def half(_):
            def chunk(g, carry):
                j0 = g * 2 * NGRP
                hga = [pltpu.async_copy(p_sh.at[sidx.at[j0 + b]], rA[b],
                                        gsemA)
                       for b in range(NGRP)]
                hgb = [pltpu.async_copy(p_sh.at[sidx.at[j0 + NGRP + b]],
                                        rB[b], gsemB)
                       for b in range(NGRP)]
                for h in hga:
                    h.wait()
                hsa = [pltpu.async_copy(rA[b], raw_sh.at[didx.at[j0 + b]],
                                        ssemA, add=True)
                       for b in range(NGRP)]
                for h in hgb:
                    h.wait()
                hsb = [pltpu.async_copy(rB[b],
                                        raw_sh.at[didx.at[j0 + NGRP + b]],
                                        ssemB, add=True)
                       for b in range(NGRP)]
                for h in hsa:
                    h.wait()
                for h in hsb:
                    h.wait()
                return carry

            lax.fori_loop(0, IDXH // (2 * NGRP), chunk, 0)

        half(0)
        pltpu.sync_copy(src_hbm.at[wid, pl.ds(IDXH, IDXH)], sidx)
        pltpu.sync_copy(dst_hbm.at[wid, pl.ds(IDXH, IDXH)], didx)
        half(1)
        plsc.subcore_barrier()
        pltpu.sync_copy(raw_sh.at[pl.ds(base, ROWS_T)],
                        out_hbm.at[c, pl.ds(base, ROWS_T)])

    if mix:
        out_type = [jax.ShapeDtypeStruct((NC, N_PAD, D), jnp.float32),
                    jax.ShapeDtypeStruct((N, D), jnp.float32)]
    else:
        out_type = jax.ShapeDtypeStruct((NC, N_PAD, D), jnp.float32)
    return functools.partial(
        pl.kernel,
        out_type=out_type,
        mesh=_mesh,
        compiler_params=pltpu.CompilerParams(use_tc_tiling_on_sc=False),
        scratch_types=(
            [pltpu.VMEM((IDXH, CH), jnp.int32),
             pltpu.VMEM((IDXH, CH), jnp.int32)]
            + [pltpu.VMEM((CH, D), jnp.float32) for _ in range(4)]
            + ([pltpu.VMEM((CH, 16), jnp.float32)] if mix else [])
            + [pltpu.VMEM_SHARED((N, D), jnp.float32),
               pltpu.VMEM_SHARED((N_PAD, D), jnp.float32),
               pltpu.SemaphoreType.DMA, pltpu.SemaphoreType.DMA,
               pltpu.SemaphoreType.DMA, pltpu.SemaphoreType.DMA]
        ),
    )(body)


_sc_step0 = _make_sc_step(False)
_sc_step_mix = _make_sc_step(True)

DEGW = 16


def _sc_deg_body(ones_hbm, dst_hbm, zeros_hbm, out_hbm,
                 didx, r0, raw_sh, ssem):
    c = lax.axis_index("c")
    s = lax.axis_index("s")
    wid = c * NS + s
    pltpu.sync_copy(dst_hbm.at[wid], didx)
    pltpu.sync_copy(ones_hbm, r0)
    base = s * ROWS_T
    pltpu.sync_copy(zeros_hbm, raw_sh.at[pl.ds(base, ROWS_T)])
    plsc.subcore_barrier()

    def chunk(g, carry):
        j0 = g * 4
        hs = [pltpu.async_copy(r0, raw_sh.at[didx.at[j0 + b]], ssem, add=True)
              for b in range(4)]
        for h in hs:
            h.wait()
        return carry

    lax.fori_loop(0, NCHUNK // 4, chunk, 0)
    plsc.subcore_barrier()
    pltpu.sync_copy(raw_sh.at[pl.ds(base, ROWS_T)],
                    out_hbm.at[c, pl.ds(base, ROWS_T)])


_sc_deg = functools.partial(
    pl.kernel,
    out_type=jax.ShapeDtypeStruct((NC, N_PAD, DEGW), jnp.float32),
    mesh=_mesh,
    compiler_params=pltpu.CompilerParams(use_tc_tiling_on_sc=False),
    scratch_types=[
        pltpu.VMEM((NCHUNK, CH), jnp.int32),
        pltpu.VMEM((CH, DEGW), jnp.float32),
        pltpu.VMEM_SHARED((N_PAD, DEGW), jnp.float32),
        pltpu.SemaphoreType.DMA,
    ],
)(_sc_deg_body)


def _tc_mlp_body(x_ref, w1_ref, b1_ref, w2_ref, b2_ref, h_ref):
    h1 = jnp.maximum(
        jnp.dot(x_ref[...], w1_ref[...], preferred_element_type=jnp.float32)
        + b1_ref[...], 0.0)
    h_ref[...] = (jnp.dot(h1, w2_ref[...], preferred_element_type=jnp.float32)
                  + b2_ref[...])


def _tc_mlp(x, W1, b1, W2, b2):
    return pl.pallas_call(
        _tc_mlp_body,
        out_shape=jax.ShapeDtypeStruct((N, D), jnp.float32),
    )(x, W1, b1, W2, b2)


def _tc_prep_body(h_ref, degp_ref, dinv_ref, p0_ref, d2_ref, hd_ref):
    deg = degp_ref[0, :N, 0:1] + degp_ref[1, :N, 0:1] + 1.0
    dinv1 = lax.rsqrt(deg)
    dinv = jnp.broadcast_to(dinv1, (N, D))
    dinv_ref[...] = dinv
    h = h_ref[...]
    p0 = dinv * h
    p0_ref[...] = p0
    d2_ref[...] = jnp.broadcast_to((1.0 - ALPHA) * dinv1 * dinv1, (N, 16))
    hd_ref[...] = ALPHA * p0


def _tc_prep(h, degp):
    return pl.pallas_call(
        _tc_prep_body,
        out_shape=[
            jax.ShapeDtypeStruct((N, D), jnp.float32),
            jax.ShapeDtypeStruct((N, D), jnp.float32),
            jax.ShapeDtypeStruct((N, 16), jnp.float32),
            jax.ShapeDtypeStruct((N, D), jnp.float32),
        ],
    )(h, degp)


def _make_tc_mix(final):
    def body(scat_ref, p_ref, dinv_ref, h_ref, o_ref):
        raw = scat_ref[0, :N, :] + scat_ref[1, :N, :] + p_ref[...]
        dinv = dinv_ref[...]
        if final:
            o_ref[...] = (1.0 - ALPHA) * dinv * raw + ALPHA * h_ref[...]
        else:
            o_ref[...] = ((1.0 - ALPHA) * dinv * dinv * raw
                          + ALPHA * dinv * h_ref[...])

    def run(scat, p, dinv, h):
        return pl.pallas_call(
            body,
            out_shape=jax.ShapeDtypeStruct((N, D), jnp.float32),
        )(scat, p, dinv, h)

    return run


_tc_mix = _make_tc_mix(False)
_tc_fin = _make_tc_mix(True)


def kernel(x, edge_index, W1, b1, W2, b2):
    src = edge_index[0]
    dst = edge_index[1]
    e = src.shape[0]
    pad = E_PAD - e
    src_p = jnp.concatenate(
        [src, jnp.zeros((pad,), jnp.int32)]).reshape(NW, NCHUNK, CH)
    dst_p = jnp.concatenate(
        [dst, jnp.full((pad,), N, jnp.int32)]).reshape(NW, NCHUNK, CH)
    zeros_h = jnp.zeros((ROWS_T, D), jnp.float32)
    ones_t = jnp.ones((CH, DEGW), jnp.float32)
    zeros16 = jnp.zeros((ROWS_T, DEGW), jnp.float32)

    degp = _sc_deg(ones_t, dst_p, zeros16)
    h = _tc_mlp(x, W1, b1.reshape(1, -1), W2, b2.reshape(1, -1))
    dinv, p, d2, hd = _tc_prep(h, degp)
    scat = _sc_step0(p, src_p, dst_p, zeros_h)
    for k in range(1, K):
        scat, p = _sc_step_mix(scat, p, d2, hd, src_p, dst_p, zeros_h)
    return _tc_fin(scat, p, dinv, h)


# speedup vs baseline: 1.0179x; 1.0179x over previous
import functools

import jax
import jax.numpy as jnp
from jax import lax
from jax.experimental import pallas as pl
from jax.experimental.pallas import tpu as pltpu
from jax.experimental.pallas import tpu_sc as plsc

N = 10000
D = 64
K = 10
ALPHA = 0.1
NC, NS = 2, 16
NW = NC * NS
CH = 128
NCHUNK = 80
EP_TILE = CH * NCHUNK
E_PAD = EP_TILE * NW
N_PAD = 10016
ROWS_T = N_PAD // NS
PROWS_T = N // NS

NGRP = 2
NSUPER = NCHUNK // (2 * NGRP)
IDXH = NCHUNK // 2

_mesh = plsc.VectorSubcoreMesh(core_axis_name="c", subcore_axis_name="s")


MIXB = 125


def _make_sc_step(mix):
    def body(*refs):
        if mix:
            (scat_hbm, pprev_hbm, d2_hbm, hd_hbm, src_hbm, dst_hbm, zeros_hbm,
             out_hbm, pnew_hbm,
             sidx, didx, rA0, rA1, rB0, rB1, d2b,
             p_sh, raw_sh, gsemA, gsemB, ssemA, ssemB) = refs
        else:
            (p_hbm, src_hbm, dst_hbm, zeros_hbm, out_hbm,
             sidx, didx, rA0, rA1, rB0, rB1,
             p_sh, raw_sh, gsemA, gsemB, ssemA, ssemB) = refs
        c = lax.axis_index("c")
        s = lax.axis_index("s")
        wid = c * NS + s
        rA = [rA0, rA1]
        rB = [rB0, rB1]
        base = s * ROWS_T
        hpro = [
            pltpu.async_copy(src_hbm.at[wid, pl.ds(0, IDXH)], sidx, ssemA),
            pltpu.async_copy(dst_hbm.at[wid, pl.ds(0, IDXH)], didx, ssemA),
            pltpu.async_copy(zeros_hbm, raw_sh.at[pl.ds(base, ROWS_T)],
                             ssemB),
        ]
        if not mix:
            pltpu.sync_copy(p_hbm.at[pl.ds(s * PROWS_T, PROWS_T)],
                            p_sh.at[pl.ds(s * PROWS_T, PROWS_T)])
        else:
            for blk in range(PROWS_T // MIXB):
                r0 = s * PROWS_T + blk * MIXB
                hs = [
                    pltpu.async_copy(scat_hbm.at[0, pl.ds(r0, MIXB)],
                                     rA0.at[pl.ds(0, MIXB)], gsemA),
                    pltpu.async_copy(scat_hbm.at[1, pl.ds(r0, MIXB)],
                                     rA1.at[pl.ds(0, MIXB)], gsemA),
                    pltpu.async_copy(pprev_hbm.at[pl.ds(r0, MIXB)],
                                     rB0.at[pl.ds(0, MIXB)], gsemA),
                    pltpu.async_copy(hd_hbm.at[pl.ds(r0, MIXB)],
                                     rB1.at[pl.ds(0, MIXB)], gsemA),
                    pltpu.async_copy(d2_hbm.at[pl.ds(r0, MIXB)],
                                     d2b.at[pl.ds(0, MIXB)], gsemA),
                ]
                for hh in hs:
                    hh.wait()

                @plsc.parallel_loop(0, MIXB, unroll=5)
                def _(r):
                    dv = d2b[r, pl.ds(0, 16)]
                    for j in range(4):
                        jj = j * 16
                        rA0[r, pl.ds(jj, 16)] = (
                            (rA0[r, pl.ds(jj, 16)] + rA1[r, pl.ds(jj, 16)]
                             + rB0[r, pl.ds(jj, 16)]) * dv
                            + rB1[r, pl.ds(jj, 16)])
                pltpu.sync_copy(rA0.at[pl.ds(0, MIXB)],
                                p_sh.at[pl.ds(r0, MIXB)])

                @pl.when(c == (blk % 2))
                def _():
                    pltpu.sync_copy(rA0.at[pl.ds(0, MIXB)],
                                    pnew_hbm.at[pl.ds(r0, MIXB)])
        for hh in hpro:
            hh.wait()
        plsc.subcore_barrier()

        def half(_):
            def chunk(g, carry):
                j0 = g * 2 * NGRP
                hga = [pltpu.async_copy(p_sh.at[sidx.at[j0 + b]], rA[b],
                                        gsemA)
                       for b in range(NGRP)]
                hgb = [pltpu.async_copy(p_sh.at[sidx.at[j0 + NGRP + b]],
                                        rB[b], gsemB)
                       for b in range(NGRP)]
                for h in hga:
                    h.wait()
                hsa = [pltpu.async_copy(rA[b], raw_sh.at[didx.at[j0 + b]],
                                        ssemA, add=True)
                       for b in range(NGRP)]
                for h in hgb:
                    h.wait()
                hsb = [pltpu.async_copy(rB[b],
                                        raw_sh.at[didx.at[j0 + NGRP + b]],
                                        ssemB, add=True)
                       for b in range(NGRP)]
                for h in hsa:
                    h.wait()
                for h in hsb:
                    h.wait()
                return carry

            lax.fori_loop(0, IDXH // (2 * NGRP), chunk, 0)

        half(0)
        pltpu.sync_copy(src_hbm.at[wid, pl.ds(IDXH, IDXH)], sidx)
        pltpu.sync_copy(dst_hbm.at[wid, pl.ds(IDXH, IDXH)], didx)
        half(1)
        plsc.subcore_barrier()
        pltpu.sync_copy(raw_sh.at[pl.ds(base, ROWS_T)],
                        out_hbm.at[c, pl.ds(base, ROWS_T)])

    if mix:
        out_type = [jax.ShapeDtypeStruct((NC, N_PAD, D), jnp.float32),
                    jax.ShapeDtypeStruct((N, D), jnp.float32)]
    else:
        out_type = jax.ShapeDtypeStruct((NC, N_PAD, D), jnp.float32)
    return functools.partial(
        pl.kernel,
        out_type=out_type,
        mesh=_mesh,
        compiler_params=pltpu.CompilerParams(use_tc_tiling_on_sc=False),
        scratch_types=(
            [pltpu.VMEM((IDXH, CH), jnp.int32),
             pltpu.VMEM((IDXH, CH), jnp.int32)]
            + [pltpu.VMEM((CH, D), jnp.float32) for _ in range(4)]
            + ([pltpu.VMEM((CH, 16), jnp.float32)] if mix else [])
            + [pltpu.VMEM_SHARED((N, D), jnp.float32),
               pltpu.VMEM_SHARED((N_PAD, D), jnp.float32),
               pltpu.SemaphoreType.DMA, pltpu.SemaphoreType.DMA,
               pltpu.SemaphoreType.DMA, pltpu.SemaphoreType.DMA]
        ),
    )(body)


_sc_step0 = _make_sc_step(False)
_sc_step_mix = _make_sc_step(True)

DEGW = 16


def _sc_deg_body(ones_hbm, dst_hbm, zeros_hbm, out_hbm,
                 didx, r0, raw_sh, ssem):
    c = lax.axis_index("c")
    s = lax.axis_index("s")
    wid = c * NS + s
    pltpu.sync_copy(dst_hbm.at[wid], didx)
    pltpu.sync_copy(ones_hbm, r0)
    base = s * ROWS_T
    pltpu.sync_copy(zeros_hbm, raw_sh.at[pl.ds(base, ROWS_T)])
    plsc.subcore_barrier()

    def chunk(g, carry):
        j0 = g * 4
        hs = [pltpu.async_copy(r0, raw_sh.at[didx.at[j0 + b]], ssem, add=True)
              for b in range(4)]
        for h in hs:
            h.wait()
        return carry

    lax.fori_loop(0, NCHUNK // 4, chunk, 0)
    plsc.subcore_barrier()
    pltpu.sync_copy(raw_sh.at[pl.ds(base, ROWS_T)],
                    out_hbm.at[c, pl.ds(base, ROWS_T)])


_sc_deg = functools.partial(
    pl.kernel,
    out_type=jax.ShapeDtypeStruct((NC, N_PAD, DEGW), jnp.float32),
    mesh=_mesh,
    compiler_params=pltpu.CompilerParams(use_tc_tiling_on_sc=False),
    scratch_types=[
        pltpu.VMEM((NCHUNK, CH), jnp.int32),
        pltpu.VMEM((CH, DEGW), jnp.float32),
        pltpu.VMEM_SHARED((N_PAD, DEGW), jnp.float32),
        pltpu.SemaphoreType.DMA,
    ],
)(_sc_deg_body)


def _tc_mlp_body(x_ref, w1_ref, b1_ref, w2_ref, b2_ref, h_ref):
    h1 = jnp.maximum(
        jnp.dot(x_ref[...], w1_ref[...], preferred_element_type=jnp.float32)
        + b1_ref[...], 0.0)
    h_ref[...] = (jnp.dot(h1, w2_ref[...], preferred_element_type=jnp.float32)
                  + b2_ref[...])


def _tc_mlp(x, W1, b1, W2, b2):
    return pl.pallas_call(
        _tc_mlp_body,
        out_shape=jax.ShapeDtypeStruct((N, D), jnp.float32),
    )(x, W1, b1, W2, b2)


def _tc_prep_body(h_ref, degp_ref, dinv_ref, p0_ref, d2_ref, hd_ref):
    deg = degp_ref[0, :N, 0:1] + degp_ref[1, :N, 0:1] + 1.0
    dinv1 = lax.rsqrt(deg)
    dinv = jnp.broadcast_to(dinv1, (N, D))
    dinv_ref[...] = dinv
    h = h_ref[...]
    p0 = dinv * h
    p0_ref[...] = p0
    d2_ref[...] = jnp.broadcast_to((1.0 - ALPHA) * dinv1 * dinv1, (N, 16))
    hd_ref[...] = ALPHA * p0


def _tc_prep(h, degp):
    return pl.pallas_call(
        _tc_prep_body,
        out_shape=[
            jax.ShapeDtypeStruct((N, D), jnp.float32),
            jax.ShapeDtypeStruct((N, D), jnp.float32),
            jax.ShapeDtypeStruct((N, 16), jnp.float32),
            jax.ShapeDtypeStruct((N, D), jnp.float32),
        ],
    )(h, degp)


def _make_tc_mix(final):
    def body(scat_ref, p_ref, dinv_ref, h_ref, o_ref):
        raw = scat_ref[0, :N, :] + scat_ref[1, :N, :] + p_ref[...]
        dinv = dinv_ref[...]
        if final:
            o_ref[...] = (1.0 - ALPHA) * dinv * raw + ALPHA * h_ref[...]
        else:
            o_ref[...] = ((1.0 - ALPHA) * dinv * dinv * raw
                          + ALPHA * dinv * h_ref[...])

    def run(scat, p, dinv, h):
        return pl.pallas_call(
            body,
            out_shape=jax.ShapeDtypeStruct((N, D), jnp.float32),
        )(scat, p, dinv, h)

    return run


_tc_mix = _make_tc_mix(False)
_tc_fin = _make_tc_mix(True)


def kernel(x, edge_index, W1, b1, W2, b2):
    src = edge_index[0]
    dst = edge_index[1]
    e = src.shape[0]
    pad = E_PAD - e
    src_p = jnp.concatenate(
        [src, jnp.zeros((pad,), jnp.int32)]).reshape(NW, NCHUNK, CH)
    dst_p = jnp.concatenate(
        [dst, jnp.full((pad,), N, jnp.int32)]).reshape(NW, NCHUNK, CH)
    zeros_h = jnp.zeros((ROWS_T, D), jnp.float32)
    ones_t = jnp.ones((CH, DEGW), jnp.float32)
    zeros16 = jnp.zeros((ROWS_T, DEGW), jnp.float32)

    degp = _sc_deg(ones_t, dst_p, zeros16)
    h = _tc_mlp(x, W1, b1.reshape(1, -1), W2, b2.reshape(1, -1))
    dinv, p, d2, hd = _tc_prep(h, degp)
    scat = _sc_step0(p, src_p, dst_p, zeros_h)
    for k in range(1, K):
        scat, p = _sc_step_mix(scat, p, d2, hd, src_p, dst_p, zeros_h)
    return _tc_fin(scat, p, dinv, h)
